# baseline (speedup 1.0000x reference)
import jax
import jax.numpy as jnp
from jax.experimental import pallas as pl

_BLK = 1024


def _blend_kernel(ids_ref, prev_ref, tab_ref, out_ref):
    sel = ids_ref[0, 0, :]
    oh_t = jnp.stack([1.0 - sel, sel], axis=0)
    emb = jax.lax.dot_general(
        oh_t, tab_ref[...], (((0,), (0,)), ((), ())),
        preferred_element_type=jnp.float32)
    out_ref[...] = prev_ref[...] + emb


def kernel(previous_embedding, token_type_ids, token_type_table):
    b, s, w = previous_embedding.shape
    n = b * s
    prev = previous_embedding.reshape(n, w)
    nb = n // _BLK
    ids = token_type_ids.reshape(nb, 1, _BLK).astype(jnp.float32)
    out = pl.pallas_call(
        _blend_kernel,
        grid=(nb,),
        in_specs=[
            pl.BlockSpec((1, 1, _BLK), lambda i: (i, 0, 0)),
            pl.BlockSpec((_BLK, w), lambda i: (i, 0)),
            pl.BlockSpec((2, w), lambda i: (0, 0)),
        ],
        out_specs=pl.BlockSpec((_BLK, w), lambda i: (i, 0)),
        out_shape=jax.ShapeDtypeStruct((n, w), jnp.float32),
    )(ids, prev, token_type_table)
    return out.reshape(b, s, w)
